# Initial kernel scaffold; baseline (speedup 1.0000x reference)
import jax
import jax.numpy as jnp
from jax.experimental import pallas as pl


def kernel(features, ins_indices_batch, ins_ids, ins_indices_len, weight, bias):
    raise NotImplementedError("write your pallas kernel here")


# speedup vs baseline: 4.0375x; 4.0375x over previous
import functools

import jax
import jax.numpy as jnp
from jax import lax
from jax.experimental import pallas as pl
from jax.experimental.pallas import tpu as pltpu

_G = 8
_EPS = 1e-5
_BLK = 2048


def _stats_tc_kernel(x_ref, idx_ref, out_ref):
    x = x_ref[...]
    idx = idx_ref[0, 0, :]
    nb, c = x.shape
    ni = out_ref.shape[0]
    onehot = (idx[:, None] == lax.broadcasted_iota(jnp.int32, (nb, ni), 1)
              ).astype(jnp.float32)
    buf = jnp.concatenate(
        [x, x * x, jnp.ones((nb, 16), jnp.float32)], axis=1)

    @pl.when(pl.program_id(0) == 0)
    def _():
        out_ref[...] = jnp.zeros_like(out_ref)

    out_ref[...] += lax.dot_general(
        onehot, buf, (((0,), (0,)), ((), ())),
        preferred_element_type=jnp.float32)


def _apply_tc_kernel(stats_ref, wb_ref, x_ref, idx_ref, out_ref):
    stats = stats_ref[...]
    ni = stats.shape[0]
    x = x_ref[...]
    nb, c = x.shape
    cpg = c // _G
    idx = idx_ref[0, 0, :]

    cnt = stats[:, 2 * c:2 * c + 1]
    denom = jnp.maximum(cnt, 1.0) * cpg
    gsel = (lax.broadcasted_iota(jnp.int32, (c, _G), 0) // cpg
            == lax.broadcasted_iota(jnp.int32, (c, _G), 1)).astype(jnp.float32)
    sum_g = jnp.dot(stats[:, :c], gsel, preferred_element_type=jnp.float32)
    sq_g = jnp.dot(stats[:, c:2 * c], gsel, preferred_element_type=jnp.float32)
    mean_g = sum_g / denom
    var_g = sq_g / denom - mean_g * mean_g
    rstd_g = lax.rsqrt(var_g + _EPS)
    mean_c = lax.dot_general(mean_g, gsel, (((1,), (1,)), ((), ())),
                             preferred_element_type=jnp.float32)
    rstd_c = lax.dot_general(rstd_g, gsel, (((1,), (1,)), ((), ())),
                             preferred_element_type=jnp.float32)
    w = wb_ref[0:1, :]
    b = wb_ref[1:2, :]
    a_coef = rstd_c * w
    b_coef = b - mean_c * a_coef

    onehot = (idx[:, None] == lax.broadcasted_iota(jnp.int32, (nb, ni), 1)
              ).astype(jnp.float32)
    a_full = jnp.dot(onehot, a_coef, preferred_element_type=jnp.float32)
    b_full = jnp.dot(onehot, b_coef, preferred_element_type=jnp.float32)
    out_ref[...] = x * a_full + b_full


def kernel(features, ins_indices_batch, ins_ids, ins_indices_len, weight, bias):
    n, c = features.shape
    ni = ins_ids.shape[0]
    nblk = n // _BLK
    idx3 = ins_indices_batch.reshape(nblk, 1, _BLK)
    sc = 2 * c + 16

    stats = pl.pallas_call(
        _stats_tc_kernel,
        grid=(nblk,),
        in_specs=[
            pl.BlockSpec((_BLK, c), lambda i: (i, 0)),
            pl.BlockSpec((1, 1, _BLK), lambda i: (i, 0, 0)),
        ],
        out_specs=pl.BlockSpec((ni, sc), lambda i: (0, 0)),
        out_shape=jax.ShapeDtypeStruct((ni, sc), jnp.float32),
        compiler_params=pltpu.CompilerParams(
            dimension_semantics=("arbitrary",)),
    )(features, idx3)

    wb = jnp.stack([weight, bias], axis=0)

    out = pl.pallas_call(
        _apply_tc_kernel,
        grid=(nblk,),
        in_specs=[
            pl.BlockSpec((ni, sc), lambda i: (0, 0)),
            pl.BlockSpec((2, c), lambda i: (0, 0)),
            pl.BlockSpec((_BLK, c), lambda i: (i, 0)),
            pl.BlockSpec((1, 1, _BLK), lambda i: (i, 0, 0)),
        ],
        out_specs=pl.BlockSpec((_BLK, c), lambda i: (i, 0)),
        out_shape=jax.ShapeDtypeStruct((n, c), jnp.float32),
        compiler_params=pltpu.CompilerParams(
            dimension_semantics=("arbitrary",)),
    )(stats, wb, features, idx3)
    return out
